# Initial kernel scaffold; baseline (speedup 1.0000x reference)
import jax
import jax.numpy as jnp
from jax.experimental import pallas as pl


def kernel(x, edge_index, W1, b1, g1, be1, W2, b2, g2, be2, Wfc, bfc):
    raise NotImplementedError("write your pallas kernel here")


# speedup vs baseline: 8.4328x; 8.4328x over previous
import functools

import jax
import jax.numpy as jnp
from jax import lax
from jax.experimental import pallas as pl
from jax.experimental.pallas import tpu as pltpu
from jax.experimental.pallas import tpu_sc as plsc

N = 10000
D = 128
E = 320000
K = 128
NBLK = E // K
NCORES = 2
NSUB = 16
HALF = N // NCORES
TRASH = 8
ACCR = HALF + TRASH
ZR = 320
LASTZ = ACCR - 15 * ZR
LASTW = HALF - 15 * ZR
FULL = NBLK // NSUB
EXTRA = NBLK - FULL * NSUB


def _mesh():
    return plsc.VectorSubcoreMesh(core_axis_name="c", subcore_axis_name="s")


def _remap(dstv, cbase):
    for j in range(K // 16):
        sl = pl.ds(j * 16, 16)
        d = dstv[sl]
        t = d - cbase
        ok = (t >= 0) & (t < HALF)
        dstv[sl] = jnp.where(ok, t, HALF + (d & (TRASH - 1)))


def _zero_acc(zbuf, acc, s):
    zero = jnp.zeros((16,), jnp.float32)

    @pl.loop(0, ZR)
    def _(r):
        for j in range(D // 16):
            zbuf[r, pl.ds(j * 16, 16)] = zero

    for i in range(15):
        @pl.when(s == i)
        def _(i=i):
            pltpu.sync_copy(zbuf, acc.at[pl.ds(i * ZR, ZR)])

    @pl.when(s == 15)
    def _():
        pltpu.sync_copy(zbuf.at[pl.ds(0, LASTZ)], acc.at[pl.ds(15 * ZR, LASTZ)])


def _write_out(acc, out_hbm, c, s):
    for i in range(15):
        @pl.when(s == i)
        def _(i=i):
            pltpu.sync_copy(acc.at[pl.ds(i * ZR, ZR)],
                            out_hbm.at[pl.ds(c * HALF + i * ZR, ZR)])

    @pl.when(s == 15)
    def _():
        pltpu.sync_copy(acc.at[pl.ds(15 * ZR, LASTW)],
                        out_hbm.at[pl.ds(c * HALF + 15 * ZR, LASTW)])


def _sc_deg(dst):
    @functools.partial(
        pl.kernel,
        out_type=jax.ShapeDtypeStruct((N, D), jnp.float32),
        mesh=_mesh(),
        scratch_types=[
            pltpu.VMEM((K,), jnp.int32),
            pltpu.VMEM((K, D), jnp.float32),
            pltpu.VMEM((ZR, D), jnp.float32),
            pltpu.VMEM_SHARED((ACCR, D), jnp.float32),
        ],
    )
    def k(dst_hbm, out_hbm, dstv, ones, zbuf, acc):
        c = lax.axis_index("c")
        s = lax.axis_index("s")
        cbase = c * HALF

        one = jnp.ones((16,), jnp.float32)

        @pl.loop(0, K)
        def _(r):
            for j in range(D // 16):
                ones[r, pl.ds(j * 16, 16)] = one

        _zero_acc(zbuf, acc, s)
        plsc.subcore_barrier()

        def do_block(b):
            pltpu.sync_copy(dst_hbm.at[pl.ds(b * K, K)], dstv)
            _remap(dstv, cbase)
            pltpu.sync_copy(ones, acc.at[dstv], add=True)

        @pl.loop(0, FULL)
        def _(i):
            do_block(s + i * NSUB)

        @pl.when(s < EXTRA)
        def _():
            do_block(FULL * NSUB + s)

        plsc.subcore_barrier()
        _write_out(acc, out_hbm, c, s)

    return k(dst)


def _sc_agg(hp, src, dst):
    @functools.partial(
        pl.kernel,
        out_type=jax.ShapeDtypeStruct((N, D), jnp.float32),
        mesh=_mesh(),
        scratch_types=[
            pltpu.VMEM((K,), jnp.int32),
            pltpu.VMEM((K,), jnp.int32),
            pltpu.VMEM((K, D), jnp.float32),
            pltpu.VMEM((ZR, D), jnp.float32),
            pltpu.VMEM_SHARED((ACCR, D), jnp.float32),
            pltpu.SemaphoreType.DMA,
        ],
    )
    def k(hp_hbm, src_hbm, dst_hbm, out_hbm, srcv, dstv, rows, zbuf, acc, sem):
        c = lax.axis_index("c")
        s = lax.axis_index("s")
        cbase = c * HALF

        _zero_acc(zbuf, acc, s)
        plsc.subcore_barrier()

        def do_block(b):
            base = b * K
            pltpu.sync_copy(src_hbm.at[pl.ds(base, K)], srcv)
            pltpu.sync_copy(dst_hbm.at[pl.ds(base, K)], dstv)
            _remap(dstv, cbase)
            pltpu.async_copy(hp_hbm.at[srcv], rows, sem).wait()
            pltpu.sync_copy(rows, acc.at[dstv], add=True)

        @pl.loop(0, FULL)
        def _(i):
            do_block(s + i * NSUB)

        @pl.when(s < EXTRA)
        def _():
            do_block(FULL * NSUB + s)

        plsc.subcore_barrier()
        _write_out(acc, out_hbm, c, s)

    return k(hp, src, dst)


def _tc_stage1(x, W1, degc):
    def body(x_ref, w_ref, deg_ref, h_ref, dinv_ref):
        deg = deg_ref[0:N, 0:1] + 1.0
        dinv = lax.rsqrt(deg)
        t = jnp.dot(x_ref[...], w_ref[...], preferred_element_type=jnp.float32)
        h_ref[...] = t * dinv
        dinv_ref[...] = dinv

    return pl.pallas_call(
        body,
        out_shape=(jax.ShapeDtypeStruct((N, D), jnp.float32),
                   jax.ShapeDtypeStruct((N, 1), jnp.float32)),
    )(x, W1, degc)


def _tc_stage2(q, hp, dinv, b, g, be, W):
    def body(q_ref, hp_ref, dinv_ref, b_ref, g_ref, be_ref, w_ref, o_ref):
        dinv = dinv_ref[...]
        z = (q_ref[...] + hp_ref[...]) * dinv + b_ref[...]
        mean = jnp.mean(z, axis=0, keepdims=True)
        zc = z - mean
        var = jnp.mean(zc * zc, axis=0, keepdims=True)
        y = g_ref[...] * zc * lax.rsqrt(var + 1e-5) + be_ref[...]
        y = jnp.maximum(y, 0.0)
        t = jnp.dot(y, w_ref[...], preferred_element_type=jnp.float32)
        o_ref[...] = t * dinv

    return pl.pallas_call(
        body,
        out_shape=jax.ShapeDtypeStruct((N, D), jnp.float32),
    )(q, hp, dinv, b, g, be, W)


def _tc_stage3(q, hp, dinv, b, g, be, Wfc, bfc):
    def body(q_ref, hp_ref, dinv_ref, b_ref, g_ref, be_ref, w_ref, bfc_ref,
             o_ref):
        z = (q_ref[...] + hp_ref[...]) * dinv_ref[...] + b_ref[...]
        mean = jnp.mean(z, axis=0, keepdims=True)
        zc = z - mean
        var = jnp.mean(zc * zc, axis=0, keepdims=True)
        y = g_ref[...] * zc * lax.rsqrt(var + 1e-5) + be_ref[...]
        y = jnp.maximum(y, 0.0)
        t = jnp.dot(y, w_ref[...], preferred_element_type=jnp.float32)
        o_ref[...] = t + bfc_ref[...]

    return pl.pallas_call(
        body,
        out_shape=jax.ShapeDtypeStruct((N, D), jnp.float32),
    )(q, hp, dinv, b, g, be, Wfc, bfc)


def kernel(x, edge_index, W1, b1, g1, be1, W2, b2, g2, be2, Wfc, bfc):
    ei = edge_index.astype(jnp.int32)
    src, dst = ei[0], ei[1]

    deg = _sc_deg(dst)
    h1p, dinv = _tc_stage1(x, W1, deg)
    q1 = _sc_agg(h1p, src, dst)
    h2p = _tc_stage2(q1, h1p, dinv, b1.reshape(1, D), g1.reshape(1, D),
                     be1.reshape(1, D), W2)
    q2 = _sc_agg(h2p, src, dst)
    return _tc_stage3(q2, h2p, dinv, b2.reshape(1, D), g2.reshape(1, D),
                      be2.reshape(1, D), Wfc, bfc.reshape(1, D))


# speedup vs baseline: 16.0450x; 1.9027x over previous
import functools

import jax
import jax.numpy as jnp
from jax import lax
from jax.experimental import pallas as pl
from jax.experimental.pallas import tpu as pltpu
from jax.experimental.pallas import tpu_sc as plsc

N = 10000
D = 128
E = 320000
K = 128
NBLK = E // K
NCORES = 2
NSUB = 16
HALF = N // NCORES
TRASH = 8
ACCR = HALF + TRASH
ZR = 320
LASTZ = ACCR - 15 * ZR
LASTW = HALF - 15 * ZR
FULL = NBLK // NSUB
EXTRA = NBLK - FULL * NSUB
PAIRS = FULL // 2


def _mesh():
    return plsc.VectorSubcoreMesh(core_axis_name="c", subcore_axis_name="s")


def _remap(dstv, cbase):
    for j in range(K // 16):
        sl = pl.ds(j * 16, 16)
        d = dstv[sl]
        t = d - cbase
        ok = (t >= 0) & (t < HALF)
        dstv[sl] = jnp.where(ok, t, HALF + (d & (TRASH - 1)))


def _zero_acc(zbuf, acc, s):
    zero = jnp.zeros((16,), jnp.float32)

    @pl.loop(0, ZR)
    def _(r):
        for j in range(D // 16):
            zbuf[r, pl.ds(j * 16, 16)] = zero

    for i in range(15):
        @pl.when(s == i)
        def _(i=i):
            pltpu.sync_copy(zbuf, acc.at[pl.ds(i * ZR, ZR)])

    @pl.when(s == 15)
    def _():
        pltpu.sync_copy(zbuf.at[pl.ds(0, LASTZ)], acc.at[pl.ds(15 * ZR, LASTZ)])


def _write_out(acc, out_hbm, c, s):
    for i in range(15):
        @pl.when(s == i)
        def _(i=i):
            pltpu.sync_copy(acc.at[pl.ds(i * ZR, ZR)],
                            out_hbm.at[pl.ds(c * HALF + i * ZR, ZR)])

    @pl.when(s == 15)
    def _():
        pltpu.sync_copy(acc.at[pl.ds(15 * ZR, LASTW)],
                        out_hbm.at[pl.ds(c * HALF + 15 * ZR, LASTW)])


def _sc_deg(dst):
    @functools.partial(
        pl.kernel,
        out_type=jax.ShapeDtypeStruct((N, D), jnp.float32),
        mesh=_mesh(),
        scratch_types=[
            pltpu.VMEM((K,), jnp.int32),
            pltpu.VMEM((K,), jnp.int32),
            pltpu.VMEM((K, D), jnp.float32),
            pltpu.VMEM((ZR, D), jnp.float32),
            pltpu.VMEM_SHARED((ACCR, D), jnp.float32),
            pltpu.SemaphoreType.DMA,
            pltpu.SemaphoreType.DMA,
        ],
    )
    def k(dst_hbm, out_hbm, dstv0, dstv1, ones, zbuf, acc, semi0, semi1):
        c = lax.axis_index("c")
        s = lax.axis_index("s")
        cbase = c * HALF

        one = jnp.ones((16,), jnp.float32)

        @pl.loop(0, K)
        def _(r):
            for j in range(D // 16):
                ones[r, pl.ds(j * 16, 16)] = one

        _zero_acc(zbuf, acc, s)
        plsc.subcore_barrier()

        def blk(i):
            return jnp.minimum(s + i * NSUB, NBLK - 1)

        def start_idx(i, dv, sem):
            return pltpu.async_copy(dst_hbm.at[pl.ds(blk(i) * K, K)], dv, sem)

        def finish(i, dv, sem, valid_tail):
            pltpu.make_async_copy(dst_hbm.at[pl.ds(0, K)], dv, sem).wait()
            _remap(dv, cbase)
            if valid_tail:
                pltpu.sync_copy(ones, acc.at[dv], add=True)
            else:
                @pl.when(s < EXTRA)
                def _():
                    pltpu.sync_copy(ones, acc.at[dv], add=True)

        start_idx(0, dstv0, semi0)

        @pl.loop(0, PAIRS)
        def _(p):
            i0 = 2 * p
            start_idx(i0 + 1, dstv1, semi1)
            finish(i0, dstv0, semi0, True)
            start_idx(i0 + 2, dstv0, semi0)
            finish(i0 + 1, dstv1, semi1, True)

        finish(FULL, dstv0, semi0, False)

        plsc.subcore_barrier()
        _write_out(acc, out_hbm, c, s)

    return k(dst)


def _sc_agg(hp, src, dst):
    @functools.partial(
        pl.kernel,
        out_type=jax.ShapeDtypeStruct((N, D), jnp.float32),
        mesh=_mesh(),
        scratch_types=[
            pltpu.VMEM((K,), jnp.int32),
            pltpu.VMEM((K,), jnp.int32),
            pltpu.VMEM((K,), jnp.int32),
            pltpu.VMEM((K,), jnp.int32),
            pltpu.VMEM((K, D), jnp.float32),
            pltpu.VMEM((K, D), jnp.float32),
            pltpu.VMEM((ZR, D), jnp.float32),
            pltpu.VMEM_SHARED((ACCR, D), jnp.float32),
            pltpu.SemaphoreType.DMA,
            pltpu.SemaphoreType.DMA,
            pltpu.SemaphoreType.DMA,
            pltpu.SemaphoreType.DMA,
        ],
    )
    def k(hp_hbm, src_hbm, dst_hbm, out_hbm, srcv0, srcv1, dstv0, dstv1,
          rows0, rows1, zbuf, acc, semi0, semi1, semg0, semg1):
        c = lax.axis_index("c")
        s = lax.axis_index("s")
        cbase = c * HALF

        _zero_acc(zbuf, acc, s)
        plsc.subcore_barrier()

        def blk(i):
            return jnp.minimum(s + i * NSUB, NBLK - 1)

        def start(i, sv, dv, rw, semi, semg):
            b = blk(i) * K
            pltpu.async_copy(src_hbm.at[pl.ds(b, K)], sv, semi)
            pltpu.async_copy(dst_hbm.at[pl.ds(b, K)], dv, semi)
            pltpu.make_async_copy(src_hbm.at[pl.ds(0, K)], sv, semi).wait()
            pltpu.make_async_copy(dst_hbm.at[pl.ds(0, K)], dv, semi).wait()
            _remap(dv, cbase)
            pltpu.async_copy(hp_hbm.at[sv], rw, semg)

        def finish(rw, dv, semg, valid_tail):
            pltpu.make_async_copy(hp_hbm.at[dv], rw, semg).wait()
            if valid_tail:
                pltpu.sync_copy(rw, acc.at[dv], add=True)
            else:
                @pl.when(s < EXTRA)
                def _():
                    pltpu.sync_copy(rw, acc.at[dv], add=True)

        start(0, srcv0, dstv0, rows0, semi0, semg0)

        @pl.loop(0, PAIRS)
        def _(p):
            i0 = 2 * p
            start(i0 + 1, srcv1, dstv1, rows1, semi1, semg1)
            finish(rows0, dstv0, semg0, True)
            start(i0 + 2, srcv0, dstv0, rows0, semi0, semg0)
            finish(rows1, dstv1, semg1, True)

        finish(rows0, dstv0, semg0, False)

        plsc.subcore_barrier()
        _write_out(acc, out_hbm, c, s)

    return k(hp, src, dst)


def _tc_stage1(x, W1, degc):
    def body(x_ref, w_ref, deg_ref, h_ref, dinv_ref):
        deg = deg_ref[0:N, 0:1] + 1.0
        dinv = lax.rsqrt(deg)
        t = jnp.dot(x_ref[...], w_ref[...], preferred_element_type=jnp.float32)
        h_ref[...] = t * dinv
        dinv_ref[...] = dinv

    return pl.pallas_call(
        body,
        out_shape=(jax.ShapeDtypeStruct((N, D), jnp.float32),
                   jax.ShapeDtypeStruct((N, 1), jnp.float32)),
    )(x, W1, degc)


def _tc_stage2(q, hp, dinv, b, g, be, W):
    def body(q_ref, hp_ref, dinv_ref, b_ref, g_ref, be_ref, w_ref, o_ref):
        dinv = dinv_ref[...]
        z = (q_ref[...] + hp_ref[...]) * dinv + b_ref[...]
        mean = jnp.mean(z, axis=0, keepdims=True)
        zc = z - mean
        var = jnp.mean(zc * zc, axis=0, keepdims=True)
        y = g_ref[...] * zc * lax.rsqrt(var + 1e-5) + be_ref[...]
        y = jnp.maximum(y, 0.0)
        t = jnp.dot(y, w_ref[...], preferred_element_type=jnp.float32)
        o_ref[...] = t * dinv

    return pl.pallas_call(
        body,
        out_shape=jax.ShapeDtypeStruct((N, D), jnp.float32),
    )(q, hp, dinv, b, g, be, W)


def _tc_stage3(q, hp, dinv, b, g, be, Wfc, bfc):
    def body(q_ref, hp_ref, dinv_ref, b_ref, g_ref, be_ref, w_ref, bfc_ref,
             o_ref):
        z = (q_ref[...] + hp_ref[...]) * dinv_ref[...] + b_ref[...]
        mean = jnp.mean(z, axis=0, keepdims=True)
        zc = z - mean
        var = jnp.mean(zc * zc, axis=0, keepdims=True)
        y = g_ref[...] * zc * lax.rsqrt(var + 1e-5) + be_ref[...]
        y = jnp.maximum(y, 0.0)
        t = jnp.dot(y, w_ref[...], preferred_element_type=jnp.float32)
        o_ref[...] = t + bfc_ref[...]

    return pl.pallas_call(
        body,
        out_shape=jax.ShapeDtypeStruct((N, D), jnp.float32),
    )(q, hp, dinv, b, g, be, Wfc, bfc)


def kernel(x, edge_index, W1, b1, g1, be1, W2, b2, g2, be2, Wfc, bfc):
    ei = edge_index.astype(jnp.int32)
    src, dst = ei[0], ei[1]

    deg = _sc_deg(dst)
    h1p, dinv = _tc_stage1(x, W1, deg)
    q1 = _sc_agg(h1p, src, dst)
    h2p = _tc_stage2(q1, h1p, dinv, b1.reshape(1, D), g1.reshape(1, D),
                     be1.reshape(1, D), W2)
    q2 = _sc_agg(h2p, src, dst)
    return _tc_stage3(q2, h2p, dinv, b2.reshape(1, D), g2.reshape(1, D),
                      be2.reshape(1, D), Wfc, bfc.reshape(1, D))


# speedup vs baseline: 16.8860x; 1.0524x over previous
import functools

import jax
import jax.numpy as jnp
from jax import lax
from jax.experimental import pallas as pl
from jax.experimental.pallas import tpu as pltpu
from jax.experimental.pallas import tpu_sc as plsc

N = 10000
D = 128
E = 320000
K = 128
NBLK = E // K
NCORES = 2
NSUB = 16
HALF = N // NCORES
TRASH = 8
ACCR = HALF + TRASH
ZR = 320
LASTZ = ACCR - 15 * ZR
LASTW = HALF - 15 * ZR
FULL = NBLK // NSUB
EXTRA = NBLK - FULL * NSUB
PAIRS = FULL // 2


def _mesh():
    return plsc.VectorSubcoreMesh(core_axis_name="c", subcore_axis_name="s")


def _remap(dstv, cbase):
    for j in range(K // 16):
        sl = pl.ds(j * 16, 16)
        d = dstv[sl]
        t = d - cbase
        ok = (t >= 0) & (t < HALF)
        dstv[sl] = jnp.where(ok, t, HALF + (d & (TRASH - 1)))


def _zero_acc(zbuf, acc, s):
    zero = jnp.zeros((16,), jnp.float32)

    @pl.loop(0, ZR)
    def _(r):
        for j in range(D // 16):
            zbuf[r, pl.ds(j * 16, 16)] = zero

    for i in range(15):
        @pl.when(s == i)
        def _(i=i):
            pltpu.sync_copy(zbuf, acc.at[pl.ds(i * ZR, ZR)])

    @pl.when(s == 15)
    def _():
        pltpu.sync_copy(zbuf.at[pl.ds(0, LASTZ)], acc.at[pl.ds(15 * ZR, LASTZ)])


def _write_out(acc, out_hbm, c, s):
    for i in range(15):
        @pl.when(s == i)
        def _(i=i):
            pltpu.sync_copy(acc.at[pl.ds(i * ZR, ZR)],
                            out_hbm.at[pl.ds(c * HALF + i * ZR, ZR)])

    @pl.when(s == 15)
    def _():
        pltpu.sync_copy(acc.at[pl.ds(15 * ZR, LASTW)],
                        out_hbm.at[pl.ds(c * HALF + 15 * ZR, LASTW)])


def _sc_deg(dst):
    @functools.partial(
        pl.kernel,
        out_type=jax.ShapeDtypeStruct((N, D), jnp.float32),
        mesh=_mesh(),
        scratch_types=(
            [pltpu.VMEM((K,), jnp.int32)] * 3 +
            [pltpu.VMEM((K, D), jnp.float32),
             pltpu.VMEM((ZR, D), jnp.float32),
             pltpu.VMEM_SHARED((ACCR, D), jnp.float32)] +
            [pltpu.SemaphoreType.DMA] * 6
        ),
    )
    def k(dst_hbm, out_hbm, dstv0, dstv1, dstv2, ones, zbuf, acc,
          semi0, semi1, semi2, sems0, sems1, sems2):
        c = lax.axis_index("c")
        s = lax.axis_index("s")
        cbase = c * HALF

        one = jnp.ones((16,), jnp.float32)
        trash = jnp.full((16,), HALF, jnp.int32)

        @pl.loop(0, K)
        def _(r):
            for j in range(D // 16):
                ones[r, pl.ds(j * 16, 16)] = one

        _zero_acc(zbuf, acc, s)
        plsc.subcore_barrier()

        slots = ((dstv0, semi0, sems0), (dstv1, semi1, sems1),
                 (dstv2, semi2, sems2))

        def blk(i):
            return jnp.minimum(s + i * NSUB, NBLK - 1)

        def vbase(i):
            ok = (i < FULL) | ((i == FULL) & (s < EXTRA))
            return cbase + jnp.where(ok, 0, 2 * N)

        def ld(i, sl):
            dv, si, _ = sl
            pltpu.async_copy(dst_hbm.at[pl.ds(blk(i) * K, K)], dv, si)
            pltpu.make_async_copy(dst_hbm.at[pl.ds(0, K)], dv, si).wait()
            _remap(dv, vbase(i))

        def scs(sl):
            dv, _, ss = sl
            pltpu.async_copy(ones, acc.at[dv], ss, add=True)

        def wsc(sl):
            dv, _, ss = sl
            pltpu.make_async_copy(ones, acc.at[dv], ss).wait()

        for j in range(K // 16):
            dstv2[pl.ds(j * 16, 16)] = trash
        scs(slots[2])
        ld(0, slots[0])
        ld(1, slots[1])

        NTRI = 53

        @pl.loop(0, NTRI)
        def _(p):
            i = 3 * p
            scs(slots[0])
            wsc(slots[2]); ld(i + 2, slots[2])
            scs(slots[1])
            wsc(slots[0]); ld(i + 3, slots[0])
            scs(slots[2])
            wsc(slots[1]); ld(i + 4, slots[1])

        wsc(slots[2])

        plsc.subcore_barrier()
        _write_out(acc, out_hbm, c, s)

    return k(dst)


def _sc_agg(hp, src, dst):
    @functools.partial(
        pl.kernel,
        out_type=jax.ShapeDtypeStruct((N, D), jnp.float32),
        mesh=_mesh(),
        scratch_types=(
            [pltpu.VMEM((K,), jnp.int32)] * 6 +
            [pltpu.VMEM((K, D), jnp.float32)] * 3 +
            [pltpu.VMEM((ZR, D), jnp.float32),
             pltpu.VMEM_SHARED((ACCR, D), jnp.float32)] +
            [pltpu.SemaphoreType.DMA] * 9
        ),
    )
    def k(hp_hbm, src_hbm, dst_hbm, out_hbm, sv0, sv1, sv2, dv0, dv1, dv2,
          rw0, rw1, rw2, zbuf, acc,
          si0, si1, si2, sg0, sg1, sg2, ss0, ss1, ss2):
        c = lax.axis_index("c")
        s = lax.axis_index("s")
        cbase = c * HALF

        trash = jnp.full((16,), HALF, jnp.int32)

        _zero_acc(zbuf, acc, s)
        plsc.subcore_barrier()

        slots = ((sv0, dv0, rw0, si0, sg0, ss0),
                 (sv1, dv1, rw1, si1, sg1, ss1),
                 (sv2, dv2, rw2, si2, sg2, ss2))

        def blk(i):
            return jnp.minimum(s + i * NSUB, NBLK - 1)

        def vbase(i):
            ok = (i < FULL) | ((i == FULL) & (s < EXTRA))
            return cbase + jnp.where(ok, 0, 2 * N)

        def ig(i, sl):
            sv, dv, rw, si, sg, _ = sl
            b = blk(i) * K
            pltpu.async_copy(src_hbm.at[pl.ds(b, K)], sv, si)
            pltpu.async_copy(dst_hbm.at[pl.ds(b, K)], dv, si)
            pltpu.make_async_copy(src_hbm.at[pl.ds(0, K)], sv, si).wait()
            pltpu.make_async_copy(dst_hbm.at[pl.ds(0, K)], dv, si).wait()
            _remap(dv, vbase(i))
            pltpu.async_copy(hp_hbm.at[sv], rw, sg)

        def wg(sl):
            sv, dv, rw, si, sg, _ = sl
            pltpu.make_async_copy(hp_hbm.at[sv], rw, sg).wait()

        def scs(sl):
            sv, dv, rw, si, sg, ss = sl
            pltpu.async_copy(rw, acc.at[dv], ss, add=True)

        def wsc(sl):
            sv, dv, rw, si, sg, ss = sl
            pltpu.make_async_copy(rw, acc.at[dv], ss).wait()

        for j in range(K // 16):
            dv2[pl.ds(j * 16, 16)] = trash
        scs(slots[2])
        ig(0, slots[0])
        ig(1, slots[1])

        NTRI = 53

        @pl.loop(0, NTRI)
        def _(p):
            i = 3 * p
            wg(slots[0]); scs(slots[0])
            wsc(slots[2]); ig(i + 2, slots[2])
            wg(slots[1]); scs(slots[1])
            wsc(slots[0]); ig(i + 3, slots[0])
            wg(slots[2]); scs(slots[2])
            wsc(slots[1]); ig(i + 4, slots[1])

        wg(slots[0])
        wg(slots[1])
        wsc(slots[2])

        plsc.subcore_barrier()
        _write_out(acc, out_hbm, c, s)

    return k(hp, src, dst)


def _tc_stage1(x, W1, degc):
    def body(x_ref, w_ref, deg_ref, h_ref, dinv_ref):
        deg = deg_ref[0:N, 0:1] + 1.0
        dinv = lax.rsqrt(deg)
        t = jnp.dot(x_ref[...], w_ref[...], preferred_element_type=jnp.float32)
        h_ref[...] = t * dinv
        dinv_ref[...] = dinv

    return pl.pallas_call(
        body,
        out_shape=(jax.ShapeDtypeStruct((N, D), jnp.float32),
                   jax.ShapeDtypeStruct((N, 1), jnp.float32)),
    )(x, W1, degc)


def _tc_stage2(q, hp, dinv, b, g, be, W):
    def body(q_ref, hp_ref, dinv_ref, b_ref, g_ref, be_ref, w_ref, o_ref):
        dinv = dinv_ref[...]
        z = (q_ref[...] + hp_ref[...]) * dinv + b_ref[...]
        mean = jnp.mean(z, axis=0, keepdims=True)
        zc = z - mean
        var = jnp.mean(zc * zc, axis=0, keepdims=True)
        y = g_ref[...] * zc * lax.rsqrt(var + 1e-5) + be_ref[...]
        y = jnp.maximum(y, 0.0)
        t = jnp.dot(y, w_ref[...], preferred_element_type=jnp.float32)
        o_ref[...] = t * dinv

    return pl.pallas_call(
        body,
        out_shape=jax.ShapeDtypeStruct((N, D), jnp.float32),
    )(q, hp, dinv, b, g, be, W)


def _tc_stage3(q, hp, dinv, b, g, be, Wfc, bfc):
    def body(q_ref, hp_ref, dinv_ref, b_ref, g_ref, be_ref, w_ref, bfc_ref,
             o_ref):
        z = (q_ref[...] + hp_ref[...]) * dinv_ref[...] + b_ref[...]
        mean = jnp.mean(z, axis=0, keepdims=True)
        zc = z - mean
        var = jnp.mean(zc * zc, axis=0, keepdims=True)
        y = g_ref[...] * zc * lax.rsqrt(var + 1e-5) + be_ref[...]
        y = jnp.maximum(y, 0.0)
        t = jnp.dot(y, w_ref[...], preferred_element_type=jnp.float32)
        o_ref[...] = t + bfc_ref[...]

    return pl.pallas_call(
        body,
        out_shape=jax.ShapeDtypeStruct((N, D), jnp.float32),
    )(q, hp, dinv, b, g, be, Wfc, bfc)


def kernel(x, edge_index, W1, b1, g1, be1, W2, b2, g2, be2, Wfc, bfc):
    ei = edge_index.astype(jnp.int32)
    src, dst = ei[0], ei[1]

    deg = _sc_deg(dst)
    h1p, dinv = _tc_stage1(x, W1, deg)
    q1 = _sc_agg(h1p, src, dst)
    h2p = _tc_stage2(q1, h1p, dinv, b1.reshape(1, D), g1.reshape(1, D),
                     be1.reshape(1, D), W2)
    q2 = _sc_agg(h2p, src, dst)
    return _tc_stage3(q2, h2p, dinv, b2.reshape(1, D), g2.reshape(1, D),
                      be2.reshape(1, D), Wfc, bfc.reshape(1, D))


# speedup vs baseline: 16.9282x; 1.0025x over previous
import functools

import jax
import jax.numpy as jnp
from jax import lax
from jax.experimental import pallas as pl
from jax.experimental.pallas import tpu as pltpu
from jax.experimental.pallas import tpu_sc as plsc

N = 10000
D = 128
E = 320000
K = 128
NBLK = E // K
NCORES = 2
NSUB = 16
HALF = N // NCORES
TRASH = 8
ACCR = HALF + TRASH
ZR = 320
LASTZ = ACCR - 15 * ZR
LASTW = HALF - 15 * ZR
FULL = NBLK // NSUB
EXTRA = NBLK - FULL * NSUB
PAIRS = FULL // 2
NTILES = NCORES * NSUB
EPT = E // NTILES
CAP = EPT + K


def _mesh():
    return plsc.VectorSubcoreMesh(core_axis_name="c", subcore_axis_name="s")


def _remap(dstv, cbase):
    for j in range(K // 16):
        sl = pl.ds(j * 16, 16)
        d = dstv[sl]
        t = d - cbase
        ok = (t >= 0) & (t < HALF)
        dstv[sl] = jnp.where(ok, t, HALF + (d & (TRASH - 1)))


def _zero_acc(zbuf, acc, s):
    zero = jnp.zeros((16,), jnp.float32)

    @pl.loop(0, ZR)
    def _(r):
        for j in range(D // 16):
            zbuf[r, pl.ds(j * 16, 16)] = zero

    for i in range(15):
        @pl.when(s == i)
        def _(i=i):
            pltpu.sync_copy(zbuf, acc.at[pl.ds(i * ZR, ZR)])

    @pl.when(s == 15)
    def _():
        pltpu.sync_copy(zbuf.at[pl.ds(0, LASTZ)], acc.at[pl.ds(15 * ZR, LASTZ)])


def _write_out(acc, out_hbm, c, s):
    for i in range(15):
        @pl.when(s == i)
        def _(i=i):
            pltpu.sync_copy(acc.at[pl.ds(i * ZR, ZR)],
                            out_hbm.at[pl.ds(c * HALF + i * ZR, ZR)])

    @pl.when(s == 15)
    def _():
        pltpu.sync_copy(acc.at[pl.ds(15 * ZR, LASTW)],
                        out_hbm.at[pl.ds(c * HALF + 15 * ZR, LASTW)])


def _sc_deg(dst):
    @functools.partial(
        pl.kernel,
        out_type=jax.ShapeDtypeStruct((N, D), jnp.float32),
        mesh=_mesh(),
        scratch_types=(
            [pltpu.VMEM((K,), jnp.int32)] * 3 +
            [pltpu.VMEM((K, D), jnp.float32),
             pltpu.VMEM((ZR, D), jnp.float32),
             pltpu.VMEM_SHARED((ACCR, D), jnp.float32)] +
            [pltpu.SemaphoreType.DMA] * 6
        ),
    )
    def k(dst_hbm, out_hbm, dstv0, dstv1, dstv2, ones, zbuf, acc,
          semi0, semi1, semi2, sems0, sems1, sems2):
        c = lax.axis_index("c")
        s = lax.axis_index("s")
        cbase = c * HALF

        one = jnp.ones((16,), jnp.float32)
        trash = jnp.full((16,), HALF, jnp.int32)

        @pl.loop(0, K)
        def _(r):
            for j in range(D // 16):
                ones[r, pl.ds(j * 16, 16)] = one

        _zero_acc(zbuf, acc, s)
        plsc.subcore_barrier()

        slots = ((dstv0, semi0, sems0), (dstv1, semi1, sems1),
                 (dstv2, semi2, sems2))

        def blk(i):
            return jnp.minimum(s + i * NSUB, NBLK - 1)

        def vbase(i):
            ok = (i < FULL) | ((i == FULL) & (s < EXTRA))
            return cbase + jnp.where(ok, 0, 2 * N)

        def ld(i, sl):
            dv, si, _ = sl
            pltpu.async_copy(dst_hbm.at[pl.ds(blk(i) * K, K)], dv, si)
            pltpu.make_async_copy(dst_hbm.at[pl.ds(0, K)], dv, si).wait()
            _remap(dv, vbase(i))

        def scs(sl):
            dv, _, ss = sl
            pltpu.async_copy(ones, acc.at[dv], ss, add=True)

        def wsc(sl):
            dv, _, ss = sl
            pltpu.make_async_copy(ones, acc.at[dv], ss).wait()

        for j in range(K // 16):
            dstv2[pl.ds(j * 16, 16)] = trash
        scs(slots[2])
        ld(0, slots[0])
        ld(1, slots[1])

        NTRI = 53

        @pl.loop(0, NTRI)
        def _(p):
            i = 3 * p
            scs(slots[0])
            wsc(slots[2]); ld(i + 2, slots[2])
            scs(slots[1])
            wsc(slots[0]); ld(i + 3, slots[0])
            scs(slots[2])
            wsc(slots[1]); ld(i + 4, slots[1])

        wsc(slots[2])

        plsc.subcore_barrier()
        _write_out(acc, out_hbm, c, s)

    return k(dst)


def _sc_agg(hp, src, dst):
    @functools.partial(
        pl.kernel,
        out_type=jax.ShapeDtypeStruct((N, D), jnp.float32),
        mesh=_mesh(),
        scratch_types=(
            [pltpu.VMEM((K,), jnp.int32)] * 6 +
            [pltpu.VMEM((K, D), jnp.float32)] * 3 +
            [pltpu.VMEM((ZR, D), jnp.float32),
             pltpu.VMEM_SHARED((ACCR, D), jnp.float32)] +
            [pltpu.SemaphoreType.DMA] * 9
        ),
    )
    def k(hp_hbm, src_hbm, dst_hbm, out_hbm, sv0, sv1, sv2, dv0, dv1, dv2,
          rw0, rw1, rw2, zbuf, acc,
          si0, si1, si2, sg0, sg1, sg2, ss0, ss1, ss2):
        c = lax.axis_index("c")
        s = lax.axis_index("s")
        cbase = c * HALF

        trash = jnp.full((16,), HALF, jnp.int32)

        _zero_acc(zbuf, acc, s)
        plsc.subcore_barrier()

        slots = ((sv0, dv0, rw0, si0, sg0, ss0),
                 (sv1, dv1, rw1, si1, sg1, ss1),
                 (sv2, dv2, rw2, si2, sg2, ss2))

        def blk(i):
            return jnp.minimum(s + i * NSUB, NBLK - 1)

        def vbase(i):
            ok = (i < FULL) | ((i == FULL) & (s < EXTRA))
            return cbase + jnp.where(ok, 0, 2 * N)

        def ig(i, sl):
            sv, dv, rw, si, sg, _ = sl
            b = blk(i) * K
            pltpu.async_copy(src_hbm.at[pl.ds(b, K)], sv, si)
            pltpu.async_copy(dst_hbm.at[pl.ds(b, K)], dv, si)
            pltpu.make_async_copy(src_hbm.at[pl.ds(0, K)], sv, si).wait()
            pltpu.make_async_copy(dst_hbm.at[pl.ds(0, K)], dv, si).wait()
            _remap(dv, vbase(i))
            pltpu.async_copy(hp_hbm.at[sv], rw, sg)

        def wg(sl):
            sv, dv, rw, si, sg, _ = sl
            pltpu.make_async_copy(hp_hbm.at[sv], rw, sg).wait()

        def scs(sl):
            sv, dv, rw, si, sg, ss = sl
            pltpu.async_copy(rw, acc.at[dv], ss, add=True)

        def wsc(sl):
            sv, dv, rw, si, sg, ss = sl
            pltpu.make_async_copy(rw, acc.at[dv], ss).wait()

        for j in range(K // 16):
            dv2[pl.ds(j * 16, 16)] = trash
        scs(slots[2])
        ig(0, slots[0])
        ig(1, slots[1])

        NTRI = 53

        @pl.loop(0, NTRI)
        def _(p):
            i = 3 * p
            wg(slots[0]); scs(slots[0])
            wsc(slots[2]); ig(i + 2, slots[2])
            wg(slots[1]); scs(slots[1])
            wsc(slots[0]); ig(i + 3, slots[0])
            wg(slots[2]); scs(slots[2])
            wsc(slots[1]); ig(i + 4, slots[1])

        wg(slots[0])
        wg(slots[1])
        wsc(slots[2])

        plsc.subcore_barrier()
        _write_out(acc, out_hbm, c, s)

    return k(hp, src, dst)


def _tc_stage1(x, W1, degc):
    def body(x_ref, w_ref, deg_ref, h_ref, dinv_ref):
        deg = deg_ref[0:N, 0:1] + 1.0
        dinv = lax.rsqrt(deg)
        t = jnp.dot(x_ref[...], w_ref[...], preferred_element_type=jnp.float32)
        h_ref[...] = t * dinv
        dinv_ref[...] = dinv

    return pl.pallas_call(
        body,
        out_shape=(jax.ShapeDtypeStruct((N, D), jnp.float32),
                   jax.ShapeDtypeStruct((N, 1), jnp.float32)),
    )(x, W1, degc)


def _tc_stage2(q, hp, dinv, b, g, be, W):
    def body(q_ref, hp_ref, dinv_ref, b_ref, g_ref, be_ref, w_ref, o_ref):
        dinv = dinv_ref[...]
        z = (q_ref[...] + hp_ref[...]) * dinv + b_ref[...]
        mean = jnp.mean(z, axis=0, keepdims=True)
        zc = z - mean
        var = jnp.mean(zc * zc, axis=0, keepdims=True)
        y = g_ref[...] * zc * lax.rsqrt(var + 1e-5) + be_ref[...]
        y = jnp.maximum(y, 0.0)
        t = jnp.dot(y, w_ref[...], preferred_element_type=jnp.float32)
        o_ref[...] = t * dinv

    return pl.pallas_call(
        body,
        out_shape=jax.ShapeDtypeStruct((N, D), jnp.float32),
    )(q, hp, dinv, b, g, be, W)


def _tc_stage3(q, hp, dinv, b, g, be, Wfc, bfc):
    def body(q_ref, hp_ref, dinv_ref, b_ref, g_ref, be_ref, w_ref, bfc_ref,
             o_ref):
        z = (q_ref[...] + hp_ref[...]) * dinv_ref[...] + b_ref[...]
        mean = jnp.mean(z, axis=0, keepdims=True)
        zc = z - mean
        var = jnp.mean(zc * zc, axis=0, keepdims=True)
        y = g_ref[...] * zc * lax.rsqrt(var + 1e-5) + be_ref[...]
        y = jnp.maximum(y, 0.0)
        t = jnp.dot(y, w_ref[...], preferred_element_type=jnp.float32)
        o_ref[...] = t + bfc_ref[...]

    return pl.pallas_call(
        body,
        out_shape=jax.ShapeDtypeStruct((N, D), jnp.float32),
    )(q, hp, dinv, b, g, be, Wfc, bfc)


def kernel(x, edge_index, W1, b1, g1, be1, W2, b2, g2, be2, Wfc, bfc):
    ei = edge_index.astype(jnp.int32)
    src, dst = ei[0], ei[1]

    deg = _sc_deg(dst)
    h1p, dinv = _tc_stage1(x, W1, deg)
    q1 = _sc_agg(h1p, src, dst)
    h2p = _tc_stage2(q1, h1p, dinv, b1.reshape(1, D), g1.reshape(1, D),
                     be1.reshape(1, D), W2)
    q2 = _sc_agg(h2p, src, dst)
    return _tc_stage3(q2, h2p, dinv, b2.reshape(1, D), g2.reshape(1, D),
                      be2.reshape(1, D), Wfc, bfc.reshape(1, D))
